# baseline (speedup 1.0000x reference)
import functools

import jax
import jax.numpy as jnp
from jax import lax
from jax.experimental import pallas as pl
from jax.experimental.pallas import tpu as pltpu
from jax.experimental.pallas import tpu_sc as plsc

_D = 128
_BS = 128
_NBUF = 2


def _sc_gather(weights, x):
    info = plsc.get_sparse_core_info()
    nw = info.num_cores * info.num_subcores
    ns, s = x.shape
    assert ns == nw * _BS
    n_chunks = s

    idx3 = x.T.reshape(s, nw, _BS).transpose(1, 0, 2)
    mesh = plsc.VectorSubcoreMesh(core_axis_name="c", subcore_axis_name="s")

    @functools.partial(
        pl.kernel,
        mesh=mesh,
        out_type=jax.ShapeDtypeStruct((s, ns, _D), jnp.float32),
        scratch_types=[
            pltpu.VMEM((n_chunks, _BS), jnp.int32),
            pltpu.VMEM((_NBUF, _BS, _D), jnp.float32),
        ] + [pltpu.SemaphoreType.DMA] * (2 * _NBUF),
    )
    def gather(table_hbm, idx_hbm, out_hbm, idx_v, rows_v, *sems):
        gs = sems[:_NBUF]
        osm = sems[_NBUF:]
        wid = lax.axis_index("s") * info.num_cores + lax.axis_index("c")
        base = wid * _BS
        pltpu.sync_copy(idx_hbm.at[wid], idx_v)

        def g_start(cc, b):
            pltpu.async_copy(
                table_hbm.at[idx_v.at[cc]], rows_v.at[b], gs[b])

        def g_wait(cc, b):
            pltpu.make_async_copy(
                table_hbm.at[idx_v.at[cc]], rows_v.at[b], gs[b]).wait()

        def o_start(cc, b):
            pltpu.async_copy(
                rows_v.at[b], out_hbm.at[cc, pl.ds(base, _BS)], osm[b])

        def o_wait(cc, b):
            pltpu.make_async_copy(
                rows_v.at[b], out_hbm.at[cc, pl.ds(base, _BS)], osm[b]).wait()

        def step(cc, b, pb, with_start, first=False):
            if not first:
                o_wait(cc - 1, pb)
            if with_start:
                g_start(cc + _NBUF - 1, pb)
            g_wait(cc, b)
            o_start(cc, b)

        for c in range(_NBUF - 1):
            g_start(c, c)
        step(0, 0, _NBUF - 1, with_start=True, first=True)

        tail_len = _NBUF + 1
        n_dyn = n_chunks - 1 - tail_len
        n_main = n_dyn // _NBUF

        def body(o, carry):
            c0 = 1 + _NBUF * o
            for db in range(_NBUF):
                step(c0 + db, (1 + db) % _NBUF, db % _NBUF, with_start=True)
            return carry

        lax.fori_loop(0, n_main, body, 0, unroll=False)
        for cc in range(1 + _NBUF * n_main, n_chunks - tail_len):
            step(cc, cc % _NBUF, (cc - 1) % _NBUF, with_start=True)

        for cc in range(n_chunks - tail_len, n_chunks):
            step(cc, cc % _NBUF, (cc - 1) % _NBUF,
                 with_start=(cc + _NBUF - 1 < n_chunks))
        o_wait(n_chunks - 1, (n_chunks - 1) % _NBUF)

    return gather(weights, idx3)


def kernel(x, weights):
    out = _sc_gather(weights, x.astype(jnp.int32))
    return out.transpose(1, 0, 2)


# speedup vs baseline: 1.0248x; 1.0248x over previous
import functools

import jax
import jax.numpy as jnp
from jax import lax
from jax.experimental import pallas as pl
from jax.experimental.pallas import tpu as pltpu
from jax.experimental.pallas import tpu_sc as plsc

_D = 128
_BS = 128
_NBUF = 4


def _sc_gather(weights, x):
    info = plsc.get_sparse_core_info()
    nw = info.num_cores * info.num_subcores
    ns, s = x.shape
    assert ns == nw * _BS
    n_chunks = s

    idx3 = x.T.reshape(s, nw, _BS).transpose(1, 0, 2)
    mesh = plsc.VectorSubcoreMesh(core_axis_name="c", subcore_axis_name="s")

    @functools.partial(
        pl.kernel,
        mesh=mesh,
        out_type=jax.ShapeDtypeStruct((s, ns, _D), jnp.float32),
        scratch_types=[
            pltpu.VMEM((n_chunks, _BS), jnp.int32),
            pltpu.VMEM((_NBUF, _BS, _D), jnp.float32),
        ] + [pltpu.SemaphoreType.DMA] * (2 * _NBUF),
    )
    def gather(table_hbm, idx_hbm, out_hbm, idx_v, rows_v, *sems):
        gs = sems[:_NBUF]
        osm = sems[_NBUF:]
        wid = lax.axis_index("s") * info.num_cores + lax.axis_index("c")
        base = wid * _BS
        pltpu.sync_copy(idx_hbm.at[wid], idx_v)

        def g_start(cc, b):
            pltpu.async_copy(
                table_hbm.at[idx_v.at[cc]], rows_v.at[b], gs[b])

        def g_wait(cc, b):
            pltpu.make_async_copy(
                table_hbm.at[idx_v.at[cc]], rows_v.at[b], gs[b]).wait()

        def o_start(cc, b):
            pltpu.async_copy(
                rows_v.at[b], out_hbm.at[cc, pl.ds(base, _BS)], osm[b])

        def o_wait(cc, b):
            pltpu.make_async_copy(
                rows_v.at[b], out_hbm.at[cc, pl.ds(base, _BS)], osm[b]).wait()

        def step(cc, b, pb, with_start, first=False):
            if not first:
                o_wait(cc - 1, pb)
            if with_start:
                g_start(cc + _NBUF - 1, pb)
            g_wait(cc, b)
            o_start(cc, b)

        for c in range(_NBUF - 1):
            g_start(c, c)
        step(0, 0, _NBUF - 1, with_start=True, first=True)

        tail_len = _NBUF + 1
        n_dyn = n_chunks - 1 - tail_len
        n_main = n_dyn // _NBUF

        def body(o, carry):
            c0 = 1 + _NBUF * o
            for db in range(_NBUF):
                step(c0 + db, (1 + db) % _NBUF, db % _NBUF, with_start=True)
            return carry

        lax.fori_loop(0, n_main, body, 0, unroll=False)
        for cc in range(1 + _NBUF * n_main, n_chunks - tail_len):
            step(cc, cc % _NBUF, (cc - 1) % _NBUF, with_start=True)

        for cc in range(n_chunks - tail_len, n_chunks):
            step(cc, cc % _NBUF, (cc - 1) % _NBUF,
                 with_start=(cc + _NBUF - 1 < n_chunks))
        o_wait(n_chunks - 1, (n_chunks - 1) % _NBUF)

    return gather(weights, idx3)


def kernel(x, weights):
    out = _sc_gather(weights, x.astype(jnp.int32))
    return out.transpose(1, 0, 2)


# speedup vs baseline: 1.0304x; 1.0055x over previous
import functools

import jax
import jax.numpy as jnp
from jax import lax
from jax.experimental import pallas as pl
from jax.experimental.pallas import tpu as pltpu
from jax.experimental.pallas import tpu_sc as plsc

_D = 128
_BS = 128
_NBUF = 6


def _sc_gather(weights, x):
    info = plsc.get_sparse_core_info()
    nw = info.num_cores * info.num_subcores
    ns, s = x.shape
    assert ns == nw * _BS
    n_chunks = s

    idx3 = x.T.reshape(s, nw, _BS).transpose(1, 0, 2)
    mesh = plsc.VectorSubcoreMesh(core_axis_name="c", subcore_axis_name="s")

    @functools.partial(
        pl.kernel,
        mesh=mesh,
        out_type=jax.ShapeDtypeStruct((s, ns, _D), jnp.float32),
        scratch_types=[
            pltpu.VMEM((n_chunks, _BS), jnp.int32),
            pltpu.VMEM((_NBUF, _BS, _D), jnp.float32),
        ] + [pltpu.SemaphoreType.DMA] * (2 * _NBUF),
    )
    def gather(table_hbm, idx_hbm, out_hbm, idx_v, rows_v, *sems):
        gs = sems[:_NBUF]
        osm = sems[_NBUF:]
        wid = lax.axis_index("s") * info.num_cores + lax.axis_index("c")
        base = wid * _BS
        pltpu.sync_copy(idx_hbm.at[wid], idx_v)

        def g_start(cc, b):
            pltpu.async_copy(
                table_hbm.at[idx_v.at[cc]], rows_v.at[b], gs[b])

        def g_wait(cc, b):
            pltpu.make_async_copy(
                table_hbm.at[idx_v.at[cc]], rows_v.at[b], gs[b]).wait()

        def o_start(cc, b):
            pltpu.async_copy(
                rows_v.at[b], out_hbm.at[cc, pl.ds(base, _BS)], osm[b])

        def o_wait(cc, b):
            pltpu.make_async_copy(
                rows_v.at[b], out_hbm.at[cc, pl.ds(base, _BS)], osm[b]).wait()

        def step(cc, b, pb, with_start, first=False):
            if not first:
                o_wait(cc - 1, pb)
            if with_start:
                g_start(cc + _NBUF - 1, pb)
            g_wait(cc, b)
            o_start(cc, b)

        for c in range(_NBUF - 1):
            g_start(c, c)
        step(0, 0, _NBUF - 1, with_start=True, first=True)

        tail_len = _NBUF + 1
        n_dyn = n_chunks - 1 - tail_len
        n_main = n_dyn // _NBUF

        def body(o, carry):
            c0 = 1 + _NBUF * o
            for db in range(_NBUF):
                step(c0 + db, (1 + db) % _NBUF, db % _NBUF, with_start=True)
            return carry

        lax.fori_loop(0, n_main, body, 0, unroll=False)
        for cc in range(1 + _NBUF * n_main, n_chunks - tail_len):
            step(cc, cc % _NBUF, (cc - 1) % _NBUF, with_start=True)

        for cc in range(n_chunks - tail_len, n_chunks):
            step(cc, cc % _NBUF, (cc - 1) % _NBUF,
                 with_start=(cc + _NBUF - 1 < n_chunks))
        o_wait(n_chunks - 1, (n_chunks - 1) % _NBUF)

    return gather(weights, idx3)


def kernel(x, weights):
    out = _sc_gather(weights, x.astype(jnp.int32))
    return out.transpose(1, 0, 2)
